# baseline (speedup 1.0000x reference)
import functools
import math

import jax
import jax.numpy as jnp
from jax import lax
from jax.experimental import pallas as pl
from jax.experimental.pallas import tpu as pltpu
from jax.experimental.pallas import tpu_sc as plsc

D = 512

_NW = 32
_RPW = 31
_LTOK = 64
_DH = 768
_NCH = _DH // 16


def _pool_sc_body(s_hbm, out_hbm, buf_v, acc_v, outv_v, sem):
    wid = lax.axis_index("s") * 2 + lax.axis_index("c")
    base = wid * _RPW

    pltpu.async_copy(s_hbm.at[base], buf_v.at[pl.ds(0, _LTOK)], sem)

    def sent_body(i, carry):
        p = lax.rem(i, 2) * _LTOK
        row = base + i
        pltpu.make_async_copy(
            s_hbm.at[row], buf_v.at[pl.ds(p, _LTOK)], sem).wait()

        @pl.when(i + 1 < _RPW)
        def _():
            pn = lax.rem(i + 1, 2) * _LTOK
            pltpu.async_copy(s_hbm.at[row + 1],
                             buf_v.at[pl.ds(pn, _LTOK)], sem)

        for c in range(_NCH):
            acc_v[pl.ds(16 * c, 16)] = buf_v[p, pl.ds(16 * c, 16)]

        def tok_body(l, c2):
            for c in range(_NCH):
                plsc.addupdate(acc_v.at[pl.ds(16 * c, 16)],
                               buf_v[p + l, pl.ds(16 * c, 16)])
            return c2
        lax.fori_loop(1, _LTOK, tok_body, 0)

        for c in range(_NCH):
            outv_v[pl.ds(i * _DH + 16 * c, 16)] = acc_v[pl.ds(16 * c, 16)]
        return carry

    lax.fori_loop(0, _RPW, sent_body, 0)
    pltpu.sync_copy(outv_v, out_hbm.at[pl.ds(base * _DH, _RPW * _DH)])


def _pool_sc(sentences_hidden3):
    BS = sentences_hidden3.shape[0]
    mesh = plsc.VectorSubcoreMesh(core_axis_name="c", subcore_axis_name="s")
    kfn = pl.kernel(
        _pool_sc_body,
        mesh=mesh,
        out_type=jax.ShapeDtypeStruct((BS * _DH,), jnp.float32),
        scratch_types=[
            pltpu.VMEM((2 * _LTOK, _DH), jnp.float32),
            pltpu.VMEM((_DH,), jnp.float32),
            pltpu.VMEM((_RPW * _DH,), jnp.float32),
            pltpu.SemaphoreType.DMA,
        ],
    )
    return kfn(sentences_hidden3).reshape(BS, _DH)


def _pool_body(s_ref, m_ref, out_ref):
    s = s_ref[...]
    m = m_ref[...]
    ps = jnp.sum(s * m[:, :, None], axis=1)
    sl = jnp.sum(m, axis=1, keepdims=True)
    sl = jnp.where(sl != 0.0, sl, 1.0)
    out_ref[...] = ps / sl


def _pool(sentences_hidden, sentences_mask, rows_per_block=16):
    BS, L, DH = sentences_hidden.shape
    nblk = BS // rows_per_block
    return pl.pallas_call(
        _pool_body,
        grid=(nblk,),
        in_specs=[
            pl.BlockSpec((rows_per_block, L, DH), lambda i: (i, 0, 0)),
            pl.BlockSpec((rows_per_block, L), lambda i: (i, 0)),
        ],
        out_specs=pl.BlockSpec((rows_per_block, DH), lambda i: (i, 0)),
        out_shape=jax.ShapeDtypeStruct((BS, DH), jnp.float32),
    )(sentences_hidden, sentences_mask)


def _dense_body(pooled_ref, mask_ref, ht_ref, nq_ref, W_hp_ref, b_hp_ref,
                W_ql_ref, b_ql_ref, g_q_ref, beta_q_ref, flag_ref, edge_ref,
                Wq_ref, Wk_ref, Wv_ref, We_ref,
                h_ref, q_ref, k_ref, v_ref, ew_ref, query_ref):
    ps = pooled_ref[...]
    sl = jnp.sum(mask_ref[...], axis=1, keepdims=True)
    inv = 1.0 / jnp.where(sl != 0.0, sl, 1.0)
    node = jnp.dot(ps, W_hp_ref[...],
                   preferred_element_type=jnp.float32) * inv + b_hp_ref[...]
    ht = ht_ref[...].astype(jnp.float32)
    f0 = flag_ref[0:1, :]
    f1 = flag_ref[1:2, :]
    h = node + f0 + ht * (f1 - f0)
    h_ref[...] = h
    q_ref[...] = jnp.dot(h, Wq_ref[...], preferred_element_type=jnp.float32)
    k_ref[...] = jnp.dot(h, Wk_ref[...], preferred_element_type=jnp.float32)
    v_ref[...] = jnp.dot(h, Wv_ref[...], preferred_element_type=jnp.float32)
    ew_ref[...] = jnp.dot(edge_ref[...], We_ref[...],
                          preferred_element_type=jnp.float32)
    ql = jnp.dot(nq_ref[...], W_ql_ref[...],
                 preferred_element_type=jnp.float32) + b_ql_ref[...]
    mu = jnp.mean(ql, axis=-1, keepdims=True)
    var = jnp.mean((ql - mu) ** 2, axis=-1, keepdims=True)
    query_ref[...] = ((ql - mu) / jnp.sqrt(var + 1e-5)) * g_q_ref[...] \
        + beta_q_ref[...]


def _dense(pooled, mask, head_flat, node_query, W_hp, b_hp, W_ql, b_ql, g_q,
           beta_q, flag_embed, edge_embed, Wq, Wk, Wv, We):
    BS, DH = pooled.shape
    B = node_query.shape[0]
    outs = (
        jax.ShapeDtypeStruct((BS, D), jnp.float32),
        jax.ShapeDtypeStruct((BS, D), jnp.float32),
        jax.ShapeDtypeStruct((BS, D), jnp.float32),
        jax.ShapeDtypeStruct((BS, D), jnp.float32),
        jax.ShapeDtypeStruct((5, D), jnp.float32),
        jax.ShapeDtypeStruct((B, D), jnp.float32),
    )
    return pl.pallas_call(_dense_body, out_shape=outs)(
        pooled, mask, head_flat, node_query, W_hp, b_hp, W_ql, b_ql, g_q,
        beta_q, flag_embed, edge_embed, Wq, Wk, Wv, We)


def _attn_body(h_ref, q_ref, k_ref, v_ref, adj_ref, et_ref, ew_ref,
               query_ref, mask_ref, W_kl_ref, b_kl_ref, g_k_ref,
               beta_k_ref, hidden_ref, recall_ref, *, S):
    h = h_ref[0]
    q = q_ref[0]
    k = k_ref[0]
    v = v_ref[0]
    adj = adj_ref[0]
    et = et_ref[0]
    dn = (((1,), (1,)), ((), ()))
    scores = lax.dot_general(q, k, dn,
                             preferred_element_type=jnp.float32)
    qe = lax.dot_general(q, ew_ref[...], dn,
                         preferred_element_type=jnp.float32)
    esc = jnp.zeros_like(scores)
    for t in range(5):
        esc = jnp.where(et == t, jnp.broadcast_to(qe[:, t:t + 1],
                                                  scores.shape), esc)
    scores = (scores + esc) * (1.0 / math.sqrt(float(D)))
    neg = jnp.float32(-1e9)
    scores = jnp.where(adj > 0, scores, neg)
    mx = jnp.max(scores, axis=-1, keepdims=True)
    p = jnp.exp(scores - mx)
    attn = p / jnp.sum(p, axis=-1, keepdims=True)
    row_has = (jnp.sum(adj.astype(jnp.float32), axis=-1, keepdims=True)
               > 0.0).astype(jnp.float32)
    attn = attn * row_has
    hidden = jnp.dot(attn, v, preferred_element_type=jnp.float32) + h
    hidden_ref[0] = hidden
    kl = jnp.dot(hidden, W_kl_ref[...],
                 preferred_element_type=jnp.float32) + b_kl_ref[...]
    mu = jnp.mean(kl, axis=-1, keepdims=True)
    var = jnp.mean((kl - mu) ** 2, axis=-1, keepdims=True)
    key = ((kl - mu) / jnp.sqrt(var + 1e-5)) * g_k_ref[...] + beta_k_ref[...]
    logits = jnp.sum(key * query_ref[0], axis=-1)
    pad = (jnp.sum(mask_ref[0], axis=-1) != 0.0).astype(jnp.float32)
    recall_ref[0] = (jax.nn.sigmoid(logits) * pad)[None, :]


def _attn(h, q, k, v, adj, et, ew, query3, mask3, W_kl, b_kl, g_k, beta_k):
    B, S, _ = h.shape
    L = mask3.shape[-1]
    bsd = pl.BlockSpec((1, S, D), lambda b: (b, 0, 0))
    bss = pl.BlockSpec((1, S, S), lambda b: (b, 0, 0))
    full = lambda shape: pl.BlockSpec(shape, lambda b: tuple(0 for _ in shape))
    outs = (
        jax.ShapeDtypeStruct((B, S, D), jnp.float32),
        jax.ShapeDtypeStruct((B, 1, S), jnp.float32),
    )
    return pl.pallas_call(
        functools.partial(_attn_body, S=S),
        grid=(B,),
        in_specs=[bsd, bsd, bsd, bsd, bss, bss,
                  full((5, D)),
                  pl.BlockSpec((1, 1, D), lambda b: (b, 0, 0)),
                  pl.BlockSpec((1, S, L), lambda b: (b, 0, 0)),
                  full((D, D)), full((1, D)), full((1, D)), full((1, D))],
        out_specs=[bsd, pl.BlockSpec((1, 1, S), lambda b: (b, 0, 0))],
        out_shape=outs,
    )(h, q, k, v, adj, et, ew, query3, mask3, W_kl, b_kl, g_k, beta_k)


def kernel(sentences_hidden, sentences_num, sentences_mask,
           sent_adjacent_matrix, head_type, edge_type, node_query,
           W_hp, b_hp, W_ql, b_ql, W_kl, b_kl, g_q, beta_q, g_k, beta_k,
           flag_embed, edge_embed, Wq, Wk, Wv, We):
    BS, L, DH = sentences_hidden.shape
    B = sentences_num.shape[0]
    S = BS // B

    pooled = _pool_sc(sentences_hidden.reshape(BS, L, DH))

    head_flat = head_type.reshape(BS, 1).astype(jnp.int32)
    r1 = lambda x: x.reshape(1, -1)
    h, q, k, v, ew, query = _dense(
        pooled, sentences_mask, head_flat, node_query, W_hp, r1(b_hp),
        W_ql, r1(b_ql), r1(g_q), r1(beta_q), flag_embed, edge_embed,
        Wq, Wk, Wv, We)

    h3 = h.reshape(B, S, D)
    q3 = q.reshape(B, S, D)
    k3 = k.reshape(B, S, D)
    v3 = v.reshape(B, S, D)
    adj = sent_adjacent_matrix.astype(jnp.int32)
    et = edge_type.astype(jnp.int32)
    mask3 = sentences_mask.reshape(B, S, L)
    hidden, recall3 = _attn(h3, q3, k3, v3, adj, et, ew,
                            query.reshape(B, 1, D), mask3,
                            W_kl, r1(b_kl), r1(g_k), r1(beta_k))
    return recall3.reshape(B, S), hidden


# speedup vs baseline: 2.5226x; 2.5226x over previous
import functools
import math

import jax
import jax.numpy as jnp
from jax import lax
from jax.experimental import pallas as pl
from jax.experimental.pallas import tpu as pltpu
from jax.experimental.pallas import tpu_sc as plsc

D = 512

_NW = 32
_RPW = 31
_LTOK = 64
_DH = 768
_NCH = _DH // 16


def _pool_sc_body(s_hbm, out_hbm, buf_v, outv_v, sem):
    wid = lax.axis_index("s") * 2 + lax.axis_index("c")
    base = wid * _RPW

    pltpu.async_copy(s_hbm.at[base], buf_v.at[pl.ds(0, _LTOK)], sem)

    def sent_body(i, carry):
        p = lax.rem(i, 2) * _LTOK
        row = base + i
        pltpu.make_async_copy(
            s_hbm.at[row], buf_v.at[pl.ds(p, _LTOK)], sem).wait()

        @pl.when(i + 1 < _RPW)
        def _():
            pn = lax.rem(i + 1, 2) * _LTOK
            pltpu.async_copy(s_hbm.at[row + 1],
                             buf_v.at[pl.ds(pn, _LTOK)], sem)

        G = 12
        for g in range(_NCH // G):
            def tok_body(l2, accs, g=g):
                l = p + 2 * l2
                a = tuple(
                    accs[j] + buf_v[l, pl.ds(16 * (G * g + j), 16)]
                    for j in range(G))
                return tuple(
                    a[j] + buf_v[l + 1, pl.ds(16 * (G * g + j), 16)]
                    for j in range(G))
            z = jnp.zeros((16,), jnp.float32)
            accs = lax.fori_loop(0, _LTOK // 2, tok_body, (z,) * G)
            for j in range(G):
                outv_v[pl.ds(i * _DH + 16 * (G * g + j), 16)] = accs[j]
        return carry

    lax.fori_loop(0, _RPW, sent_body, 0)
    pltpu.sync_copy(outv_v, out_hbm.at[pl.ds(base * _DH, _RPW * _DH)])


def _pool_sc(sentences_hidden3):
    BS = sentences_hidden3.shape[0]
    mesh = plsc.VectorSubcoreMesh(core_axis_name="c", subcore_axis_name="s")
    kfn = pl.kernel(
        _pool_sc_body,
        mesh=mesh,
        out_type=jax.ShapeDtypeStruct((BS * _DH,), jnp.float32),
        scratch_types=[
            pltpu.VMEM((2 * _LTOK, _DH), jnp.float32),
            pltpu.VMEM((_RPW * _DH,), jnp.float32),
            pltpu.SemaphoreType.DMA,
        ],
    )
    return kfn(sentences_hidden3).reshape(BS, _DH)


def _pool_body(s_ref, m_ref, out_ref):
    s = s_ref[...]
    m = m_ref[...]
    ps = jnp.sum(s * m[:, :, None], axis=1)
    sl = jnp.sum(m, axis=1, keepdims=True)
    sl = jnp.where(sl != 0.0, sl, 1.0)
    out_ref[...] = ps / sl


def _pool(sentences_hidden, sentences_mask, rows_per_block=16):
    BS, L, DH = sentences_hidden.shape
    nblk = BS // rows_per_block
    return pl.pallas_call(
        _pool_body,
        grid=(nblk,),
        in_specs=[
            pl.BlockSpec((rows_per_block, L, DH), lambda i: (i, 0, 0)),
            pl.BlockSpec((rows_per_block, L), lambda i: (i, 0)),
        ],
        out_specs=pl.BlockSpec((rows_per_block, DH), lambda i: (i, 0)),
        out_shape=jax.ShapeDtypeStruct((BS, DH), jnp.float32),
    )(sentences_hidden, sentences_mask)


def _dense_body(pooled_ref, mask_ref, ht_ref, nq_ref, W_hp_ref, b_hp_ref,
                W_ql_ref, b_ql_ref, g_q_ref, beta_q_ref, flag_ref, edge_ref,
                Wq_ref, Wk_ref, Wv_ref, We_ref,
                h_ref, q_ref, k_ref, v_ref, ew_ref, query_ref):
    ps = pooled_ref[...]
    sl = jnp.sum(mask_ref[...], axis=1, keepdims=True)
    inv = 1.0 / jnp.where(sl != 0.0, sl, 1.0)
    node = jnp.dot(ps, W_hp_ref[...],
                   preferred_element_type=jnp.float32) * inv + b_hp_ref[...]
    ht = ht_ref[...].astype(jnp.float32)
    f0 = flag_ref[0:1, :]
    f1 = flag_ref[1:2, :]
    h = node + f0 + ht * (f1 - f0)
    h_ref[...] = h
    q_ref[...] = jnp.dot(h, Wq_ref[...], preferred_element_type=jnp.float32)
    k_ref[...] = jnp.dot(h, Wk_ref[...], preferred_element_type=jnp.float32)
    v_ref[...] = jnp.dot(h, Wv_ref[...], preferred_element_type=jnp.float32)
    ew_ref[...] = jnp.dot(edge_ref[...], We_ref[...],
                          preferred_element_type=jnp.float32)
    ql = jnp.dot(nq_ref[...], W_ql_ref[...],
                 preferred_element_type=jnp.float32) + b_ql_ref[...]
    mu = jnp.mean(ql, axis=-1, keepdims=True)
    var = jnp.mean((ql - mu) ** 2, axis=-1, keepdims=True)
    query_ref[...] = ((ql - mu) / jnp.sqrt(var + 1e-5)) * g_q_ref[...] \
        + beta_q_ref[...]


def _dense(pooled, mask, head_flat, node_query, W_hp, b_hp, W_ql, b_ql, g_q,
           beta_q, flag_embed, edge_embed, Wq, Wk, Wv, We):
    BS, DH = pooled.shape
    B = node_query.shape[0]
    outs = (
        jax.ShapeDtypeStruct((BS, D), jnp.float32),
        jax.ShapeDtypeStruct((BS, D), jnp.float32),
        jax.ShapeDtypeStruct((BS, D), jnp.float32),
        jax.ShapeDtypeStruct((BS, D), jnp.float32),
        jax.ShapeDtypeStruct((5, D), jnp.float32),
        jax.ShapeDtypeStruct((B, D), jnp.float32),
    )
    return pl.pallas_call(_dense_body, out_shape=outs)(
        pooled, mask, head_flat, node_query, W_hp, b_hp, W_ql, b_ql, g_q,
        beta_q, flag_embed, edge_embed, Wq, Wk, Wv, We)


def _attn_body(h_ref, q_ref, k_ref, v_ref, adj_ref, et_ref, ew_ref,
               query_ref, mask_ref, W_kl_ref, b_kl_ref, g_k_ref,
               beta_k_ref, hidden_ref, recall_ref, *, S):
    h = h_ref[0]
    q = q_ref[0]
    k = k_ref[0]
    v = v_ref[0]
    adj = adj_ref[0]
    et = et_ref[0]
    dn = (((1,), (1,)), ((), ()))
    scores = lax.dot_general(q, k, dn,
                             preferred_element_type=jnp.float32)
    qe = lax.dot_general(q, ew_ref[...], dn,
                         preferred_element_type=jnp.float32)
    esc = jnp.zeros_like(scores)
    for t in range(5):
        esc = jnp.where(et == t, jnp.broadcast_to(qe[:, t:t + 1],
                                                  scores.shape), esc)
    scores = (scores + esc) * (1.0 / math.sqrt(float(D)))
    neg = jnp.float32(-1e9)
    scores = jnp.where(adj > 0, scores, neg)
    mx = jnp.max(scores, axis=-1, keepdims=True)
    p = jnp.exp(scores - mx)
    attn = p / jnp.sum(p, axis=-1, keepdims=True)
    row_has = (jnp.sum(adj.astype(jnp.float32), axis=-1, keepdims=True)
               > 0.0).astype(jnp.float32)
    attn = attn * row_has
    hidden = jnp.dot(attn, v, preferred_element_type=jnp.float32) + h
    hidden_ref[0] = hidden
    kl = jnp.dot(hidden, W_kl_ref[...],
                 preferred_element_type=jnp.float32) + b_kl_ref[...]
    mu = jnp.mean(kl, axis=-1, keepdims=True)
    var = jnp.mean((kl - mu) ** 2, axis=-1, keepdims=True)
    key = ((kl - mu) / jnp.sqrt(var + 1e-5)) * g_k_ref[...] + beta_k_ref[...]
    logits = jnp.sum(key * query_ref[0], axis=-1)
    pad = (jnp.sum(mask_ref[0], axis=-1) != 0.0).astype(jnp.float32)
    recall_ref[0] = (jax.nn.sigmoid(logits) * pad)[None, :]


def _attn(h, q, k, v, adj, et, ew, query3, mask3, W_kl, b_kl, g_k, beta_k):
    B, S, _ = h.shape
    L = mask3.shape[-1]
    bsd = pl.BlockSpec((1, S, D), lambda b: (b, 0, 0))
    bss = pl.BlockSpec((1, S, S), lambda b: (b, 0, 0))
    full = lambda shape: pl.BlockSpec(shape, lambda b: tuple(0 for _ in shape))
    outs = (
        jax.ShapeDtypeStruct((B, S, D), jnp.float32),
        jax.ShapeDtypeStruct((B, 1, S), jnp.float32),
    )
    return pl.pallas_call(
        functools.partial(_attn_body, S=S),
        grid=(B,),
        in_specs=[bsd, bsd, bsd, bsd, bss, bss,
                  full((5, D)),
                  pl.BlockSpec((1, 1, D), lambda b: (b, 0, 0)),
                  pl.BlockSpec((1, S, L), lambda b: (b, 0, 0)),
                  full((D, D)), full((1, D)), full((1, D)), full((1, D))],
        out_specs=[bsd, pl.BlockSpec((1, 1, S), lambda b: (b, 0, 0))],
        out_shape=outs,
    )(h, q, k, v, adj, et, ew, query3, mask3, W_kl, b_kl, g_k, beta_k)


def kernel(sentences_hidden, sentences_num, sentences_mask,
           sent_adjacent_matrix, head_type, edge_type, node_query,
           W_hp, b_hp, W_ql, b_ql, W_kl, b_kl, g_q, beta_q, g_k, beta_k,
           flag_embed, edge_embed, Wq, Wk, Wv, We):
    BS, L, DH = sentences_hidden.shape
    B = sentences_num.shape[0]
    S = BS // B

    pooled = _pool_sc(sentences_hidden.reshape(BS, L, DH))

    head_flat = head_type.reshape(BS, 1).astype(jnp.int32)
    r1 = lambda x: x.reshape(1, -1)
    h, q, k, v, ew, query = _dense(
        pooled, sentences_mask, head_flat, node_query, W_hp, r1(b_hp),
        W_ql, r1(b_ql), r1(g_q), r1(beta_q), flag_embed, edge_embed,
        Wq, Wk, Wv, We)

    h3 = h.reshape(B, S, D)
    q3 = q.reshape(B, S, D)
    k3 = k.reshape(B, S, D)
    v3 = v.reshape(B, S, D)
    adj = sent_adjacent_matrix.astype(jnp.int32)
    et = edge_type.astype(jnp.int32)
    mask3 = sentences_mask.reshape(B, S, L)
    hidden, recall3 = _attn(h3, q3, k3, v3, adj, et, ew,
                            query.reshape(B, 1, D), mask3,
                            W_kl, r1(b_kl), r1(g_k), r1(beta_k))
    return recall3.reshape(B, S), hidden


# speedup vs baseline: 2.9418x; 1.1662x over previous
import functools
import math

import jax
import jax.numpy as jnp
from jax import lax
from jax.experimental import pallas as pl
from jax.experimental.pallas import tpu as pltpu
from jax.experimental.pallas import tpu_sc as plsc

D = 512

_NW = 32
_R_SC = 512
_RPW = _R_SC // _NW
_LTOK = 64
_DH = 768
_NCH = _DH // 16


def _pool_sc_body(s_hbm, out_hbm, buf_v, outv_v, sem):
    wid = lax.axis_index("s") * 2 + lax.axis_index("c")
    base = wid * _RPW

    pltpu.async_copy(s_hbm.at[base], buf_v.at[pl.ds(0, _LTOK)], sem)

    def sent_body(i, carry):
        p = lax.rem(i, 2) * _LTOK
        row = base + i
        pltpu.make_async_copy(
            s_hbm.at[row], buf_v.at[pl.ds(p, _LTOK)], sem).wait()

        @pl.when(i + 1 < _RPW)
        def _():
            pn = lax.rem(i + 1, 2) * _LTOK
            pltpu.async_copy(s_hbm.at[row + 1],
                             buf_v.at[pl.ds(pn, _LTOK)], sem)

        G = 12
        for g in range(_NCH // G):
            def tok_body(l2, accs, g=g):
                l = p + 2 * l2
                a = tuple(
                    accs[j] + buf_v[l, pl.ds(16 * (G * g + j), 16)]
                    for j in range(G))
                return tuple(
                    a[j] + buf_v[l + 1, pl.ds(16 * (G * g + j), 16)]
                    for j in range(G))
            z = jnp.zeros((16,), jnp.float32)
            accs = lax.fori_loop(0, _LTOK // 2, tok_body, (z,) * G)
            for j in range(G):
                outv_v[pl.ds(i * _DH + 16 * (G * g + j), 16)] = accs[j]
        return carry

    lax.fori_loop(0, _RPW, sent_body, 0)
    pltpu.sync_copy(outv_v, out_hbm.at[pl.ds(base * _DH, _RPW * _DH)])


def _pool_sc(sentences_hidden3):
    mesh = plsc.VectorSubcoreMesh(core_axis_name="c", subcore_axis_name="s")
    kfn = pl.kernel(
        _pool_sc_body,
        mesh=mesh,
        out_type=jax.ShapeDtypeStruct((_R_SC * _DH,), jnp.float32),
        scratch_types=[
            pltpu.VMEM((2 * _LTOK, _DH), jnp.float32),
            pltpu.VMEM((_RPW * _DH,), jnp.float32),
            pltpu.SemaphoreType.DMA,
        ],
    )
    return kfn(sentences_hidden3).reshape(_R_SC, _DH)


def _pool_body(s_ref, out_ref):
    out_ref[...] = jnp.sum(s_ref[...], axis=1)


def _pool_tc_tail(sentences_hidden3, rows_per_block=16):
    BS, L, DH = sentences_hidden3.shape
    ntail = BS - _R_SC
    nblk = ntail // rows_per_block
    off = _R_SC // rows_per_block
    return pl.pallas_call(
        _pool_body,
        grid=(nblk,),
        in_specs=[
            pl.BlockSpec((rows_per_block, L, DH), lambda i: (i + off, 0, 0)),
        ],
        out_specs=pl.BlockSpec((rows_per_block, DH), lambda i: (i, 0)),
        out_shape=jax.ShapeDtypeStruct((ntail, DH), jnp.float32),
    )(sentences_hidden3)


def _dense_body(ps_sc_ref, ps_tc_ref, mask_ref, ht_ref, nq_ref, W_hp_ref,
                b_hp_ref, W_ql_ref, b_ql_ref, g_q_ref, beta_q_ref, flag_ref,
                edge_ref, Wq_ref, Wk_ref, Wv_ref, We_ref,
                h_ref, q_ref, k_ref, v_ref, ew_ref, query_ref):
    ps = jnp.concatenate([ps_sc_ref[...], ps_tc_ref[...]], axis=0)
    sl = jnp.sum(mask_ref[...], axis=1, keepdims=True)
    inv = 1.0 / jnp.where(sl != 0.0, sl, 1.0)
    node = jnp.dot(ps, W_hp_ref[...],
                   preferred_element_type=jnp.float32) * inv + b_hp_ref[...]
    ht = ht_ref[...].astype(jnp.float32)
    f0 = flag_ref[0:1, :]
    f1 = flag_ref[1:2, :]
    h = node + f0 + ht * (f1 - f0)
    h_ref[...] = h
    q_ref[...] = jnp.dot(h, Wq_ref[...], preferred_element_type=jnp.float32)
    k_ref[...] = jnp.dot(h, Wk_ref[...], preferred_element_type=jnp.float32)
    v_ref[...] = jnp.dot(h, Wv_ref[...], preferred_element_type=jnp.float32)
    ew_ref[...] = jnp.dot(edge_ref[...], We_ref[...],
                          preferred_element_type=jnp.float32)
    ql = jnp.dot(nq_ref[...], W_ql_ref[...],
                 preferred_element_type=jnp.float32) + b_ql_ref[...]
    mu = jnp.mean(ql, axis=-1, keepdims=True)
    var = jnp.mean((ql - mu) ** 2, axis=-1, keepdims=True)
    query_ref[...] = ((ql - mu) / jnp.sqrt(var + 1e-5)) * g_q_ref[...] \
        + beta_q_ref[...]


def _dense(ps_sc, ps_tc, mask, head_flat, node_query, W_hp, b_hp, W_ql,
           b_ql, g_q, beta_q, flag_embed, edge_embed, Wq, Wk, Wv, We):
    BS = ps_sc.shape[0] + ps_tc.shape[0]
    B = node_query.shape[0]
    outs = (
        jax.ShapeDtypeStruct((BS, D), jnp.float32),
        jax.ShapeDtypeStruct((BS, D), jnp.float32),
        jax.ShapeDtypeStruct((BS, D), jnp.float32),
        jax.ShapeDtypeStruct((BS, D), jnp.float32),
        jax.ShapeDtypeStruct((5, D), jnp.float32),
        jax.ShapeDtypeStruct((B, D), jnp.float32),
    )
    return pl.pallas_call(_dense_body, out_shape=outs)(
        ps_sc, ps_tc, mask, head_flat, node_query, W_hp, b_hp, W_ql, b_ql,
        g_q, beta_q, flag_embed, edge_embed, Wq, Wk, Wv, We)


def _attn_body(h_ref, q_ref, k_ref, v_ref, adj_ref, et_ref, ew_ref,
               query_ref, mask_ref, W_kl_ref, b_kl_ref, g_k_ref,
               beta_k_ref, hidden_ref, recall_ref, *, S):
    h = h_ref[0]
    q = q_ref[0]
    k = k_ref[0]
    v = v_ref[0]
    adj = adj_ref[0]
    et = et_ref[0]
    dn = (((1,), (1,)), ((), ()))
    scores = lax.dot_general(q, k, dn,
                             preferred_element_type=jnp.float32)
    qe = lax.dot_general(q, ew_ref[...], dn,
                         preferred_element_type=jnp.float32)
    esc = jnp.zeros_like(scores)
    for t in range(5):
        esc = jnp.where(et == t, jnp.broadcast_to(qe[:, t:t + 1],
                                                  scores.shape), esc)
    scores = (scores + esc) * (1.0 / math.sqrt(float(D)))
    neg = jnp.float32(-1e9)
    scores = jnp.where(adj > 0, scores, neg)
    mx = jnp.max(scores, axis=-1, keepdims=True)
    p = jnp.exp(scores - mx)
    attn = p / jnp.sum(p, axis=-1, keepdims=True)
    row_has = (jnp.sum(adj.astype(jnp.float32), axis=-1, keepdims=True)
               > 0.0).astype(jnp.float32)
    attn = attn * row_has
    hidden = jnp.dot(attn, v, preferred_element_type=jnp.float32) + h
    hidden_ref[0] = hidden
    kl = jnp.dot(hidden, W_kl_ref[...],
                 preferred_element_type=jnp.float32) + b_kl_ref[...]
    mu = jnp.mean(kl, axis=-1, keepdims=True)
    var = jnp.mean((kl - mu) ** 2, axis=-1, keepdims=True)
    key = ((kl - mu) / jnp.sqrt(var + 1e-5)) * g_k_ref[...] + beta_k_ref[...]
    logits = jnp.sum(key * query_ref[0], axis=-1)
    pad = (jnp.sum(mask_ref[0], axis=-1) != 0.0).astype(jnp.float32)
    recall_ref[0] = (jax.nn.sigmoid(logits) * pad)[None, :]


def _attn(h, q, k, v, adj, et, ew, query3, mask3, W_kl, b_kl, g_k, beta_k):
    B, S, _ = h.shape
    L = mask3.shape[-1]
    bsd = pl.BlockSpec((1, S, D), lambda b: (b, 0, 0))
    bss = pl.BlockSpec((1, S, S), lambda b: (b, 0, 0))
    full = lambda shape: pl.BlockSpec(shape, lambda b: tuple(0 for _ in shape))
    outs = (
        jax.ShapeDtypeStruct((B, S, D), jnp.float32),
        jax.ShapeDtypeStruct((B, 1, S), jnp.float32),
    )
    return pl.pallas_call(
        functools.partial(_attn_body, S=S),
        grid=(B,),
        in_specs=[bsd, bsd, bsd, bsd, bss, bss,
                  full((5, D)),
                  pl.BlockSpec((1, 1, D), lambda b: (b, 0, 0)),
                  pl.BlockSpec((1, S, L), lambda b: (b, 0, 0)),
                  full((D, D)), full((1, D)), full((1, D)), full((1, D))],
        out_specs=[bsd, pl.BlockSpec((1, 1, S), lambda b: (b, 0, 0))],
        out_shape=outs,
    )(h, q, k, v, adj, et, ew, query3, mask3, W_kl, b_kl, g_k, beta_k)


def kernel(sentences_hidden, sentences_num, sentences_mask,
           sent_adjacent_matrix, head_type, edge_type, node_query,
           W_hp, b_hp, W_ql, b_ql, W_kl, b_kl, g_q, beta_q, g_k, beta_k,
           flag_embed, edge_embed, Wq, Wk, Wv, We):
    BS, L, DH = sentences_hidden.shape
    B = sentences_num.shape[0]
    S = BS // B

    s3 = sentences_hidden.reshape(BS, L, DH)
    ps_sc = _pool_sc(s3)
    ps_tc = _pool_tc_tail(s3)

    head_flat = head_type.reshape(BS, 1).astype(jnp.int32)
    r1 = lambda x: x.reshape(1, -1)
    h, q, k, v, ew, query = _dense(
        ps_sc, ps_tc, sentences_mask, head_flat, node_query, W_hp, r1(b_hp),
        W_ql, r1(b_ql), r1(g_q), r1(beta_q), flag_embed, edge_embed,
        Wq, Wk, Wv, We)

    h3 = h.reshape(B, S, D)
    q3 = q.reshape(B, S, D)
    k3 = k.reshape(B, S, D)
    v3 = v.reshape(B, S, D)
    adj = sent_adjacent_matrix.astype(jnp.int32)
    et = edge_type.astype(jnp.int32)
    mask3 = sentences_mask.reshape(B, S, L)
    hidden, recall3 = _attn(h3, q3, k3, v3, adj, et, ew,
                            query.reshape(B, 1, D), mask3,
                            W_kl, r1(b_kl), r1(g_k), r1(beta_k))
    return recall3.reshape(B, S), hidden


# speedup vs baseline: 3.7344x; 1.2694x over previous
import functools
import math

import jax
import jax.numpy as jnp
from jax import lax
from jax.experimental import pallas as pl
from jax.experimental.pallas import tpu as pltpu
from jax.experimental.pallas import tpu_sc as plsc

D = 512

_NW = 32
_R_SC = 512
_RPW = _R_SC // _NW
_LTOK = 64
_DH = 768
_NCH = _DH // 16


def _pool_sc_body(s_hbm, out_hbm, buf_v, outv_v, sem):
    wid = lax.axis_index("s") * 2 + lax.axis_index("c")
    base = wid * _RPW

    pltpu.async_copy(s_hbm.at[base], buf_v.at[pl.ds(0, _LTOK)], sem)

    def sent_body(i, carry):
        p = lax.rem(i, 2) * _LTOK
        row = base + i
        pltpu.make_async_copy(
            s_hbm.at[row], buf_v.at[pl.ds(p, _LTOK)], sem).wait()

        @pl.when(i + 1 < _RPW)
        def _():
            pn = lax.rem(i + 1, 2) * _LTOK
            pltpu.async_copy(s_hbm.at[row + 1],
                             buf_v.at[pl.ds(pn, _LTOK)], sem)

        G = 12
        for g in range(_NCH // G):
            def tok_body(l2, accs, g=g):
                l = p + 2 * l2
                a = tuple(
                    accs[j] + buf_v[l, pl.ds(16 * (G * g + j), 16)]
                    for j in range(G))
                return tuple(
                    a[j] + buf_v[l + 1, pl.ds(16 * (G * g + j), 16)]
                    for j in range(G))
            z = jnp.zeros((16,), jnp.float32)
            accs = lax.fori_loop(0, _LTOK // 2, tok_body, (z,) * G)
            for j in range(G):
                outv_v[pl.ds(i * _DH + 16 * (G * g + j), 16)] = accs[j]
        return carry

    lax.fori_loop(0, _RPW, sent_body, 0)
    pltpu.sync_copy(outv_v, out_hbm.at[pl.ds(base * _DH, _RPW * _DH)])


def _pool_sc(sentences_hidden3):
    mesh = plsc.VectorSubcoreMesh(core_axis_name="c", subcore_axis_name="s")
    kfn = pl.kernel(
        _pool_sc_body,
        mesh=mesh,
        out_type=jax.ShapeDtypeStruct((_R_SC * _DH,), jnp.float32),
        scratch_types=[
            pltpu.VMEM((2 * _LTOK, _DH), jnp.float32),
            pltpu.VMEM((_RPW * _DH,), jnp.float32),
            pltpu.SemaphoreType.DMA,
        ],
    )
    return kfn(sentences_hidden3).reshape(_R_SC, _DH)


def _pool_body(s_ref, out_ref):
    out_ref[...] = jnp.sum(s_ref[...], axis=1)


def _pool_tc_tail(sentences_hidden3, rows_per_block=16):
    BS, L, DH = sentences_hidden3.shape
    ntail = BS - _R_SC
    nblk = ntail // rows_per_block
    off = _R_SC // rows_per_block
    return pl.pallas_call(
        _pool_body,
        grid=(nblk,),
        in_specs=[
            pl.BlockSpec((rows_per_block, L, DH), lambda i: (i + off, 0, 0)),
        ],
        out_specs=pl.BlockSpec((rows_per_block, DH), lambda i: (i, 0)),
        out_shape=jax.ShapeDtypeStruct((ntail, DH), jnp.float32),
    )(sentences_hidden3)


def _dense_body(ps_sc_ref, ps_tc_ref, mask_ref, ht_ref, nq_ref, W_hp_ref,
                b_hp_ref, W_ql_ref, b_ql_ref, g_q_ref, beta_q_ref, flag_ref,
                edge_ref, Wq_ref, Wk_ref, Wv_ref, We_ref,
                h_ref, q_ref, k_ref, v_ref, ew_ref, query_ref):
    ps = jnp.concatenate([ps_sc_ref[...], ps_tc_ref[...]], axis=0)
    sl = jnp.sum(mask_ref[...], axis=1, keepdims=True)
    inv = 1.0 / jnp.where(sl != 0.0, sl, 1.0)
    node = jnp.dot(ps, W_hp_ref[...],
                   preferred_element_type=jnp.float32) * inv + b_hp_ref[...]
    ht = ht_ref[...].astype(jnp.float32)
    f0 = flag_ref[0:1, :]
    f1 = flag_ref[1:2, :]
    h = node + f0 + ht * (f1 - f0)
    h_ref[...] = h
    q_ref[...] = jnp.dot(h, Wq_ref[...], preferred_element_type=jnp.float32)
    k_ref[...] = jnp.dot(h, Wk_ref[...], preferred_element_type=jnp.float32)
    v_ref[...] = jnp.dot(h, Wv_ref[...], preferred_element_type=jnp.float32)
    ew_ref[...] = jnp.dot(edge_ref[...], We_ref[...],
                          preferred_element_type=jnp.float32)
    ql = jnp.dot(nq_ref[...], W_ql_ref[...],
                 preferred_element_type=jnp.float32) + b_ql_ref[...]
    mu = jnp.mean(ql, axis=-1, keepdims=True)
    var = jnp.mean((ql - mu) ** 2, axis=-1, keepdims=True)
    query_ref[...] = ((ql - mu) / jnp.sqrt(var + 1e-5)) * g_q_ref[...] \
        + beta_q_ref[...]


def _dense(ps_sc, ps_tc, mask, head_flat, node_query, W_hp, b_hp, W_ql,
           b_ql, g_q, beta_q, flag_embed, edge_embed, Wq, Wk, Wv, We):
    BS = ps_sc.shape[0] + ps_tc.shape[0]
    B = node_query.shape[0]
    outs = (
        jax.ShapeDtypeStruct((BS, D), jnp.float32),
        jax.ShapeDtypeStruct((BS, D), jnp.float32),
        jax.ShapeDtypeStruct((BS, D), jnp.float32),
        jax.ShapeDtypeStruct((BS, D), jnp.float32),
        jax.ShapeDtypeStruct((5, D), jnp.float32),
        jax.ShapeDtypeStruct((B, D), jnp.float32),
    )
    return pl.pallas_call(_dense_body, out_shape=outs)(
        ps_sc, ps_tc, mask, head_flat, node_query, W_hp, b_hp, W_ql, b_ql,
        g_q, beta_q, flag_embed, edge_embed, Wq, Wk, Wv, We)


_GPB = 8


def _attn_body(h_ref, q_ref, k_ref, v_ref, adj_ref, et_ref, ew_ref,
               query_ref, mask_ref, W_kl_ref, b_kl_ref, g_k_ref,
               beta_k_ref, hidden_ref, recall_ref, *, S):
    dn = (((1,), (1,)), ((), ()))
    qall = q_ref[...]
    qe = lax.dot_general(qall, ew_ref[...], dn,
                         preferred_element_type=jnp.float32)
    isq = 1.0 / math.sqrt(float(D))
    neg = jnp.float32(-1e9)
    outs = []
    for j in range(_GPB):
        sl = slice(j * S, (j + 1) * S)
        q = qall[sl]
        k = k_ref[sl]
        v = v_ref[sl]
        adj = adj_ref[j]
        et = et_ref[j]
        scores = lax.dot_general(q, k, dn,
                                 preferred_element_type=jnp.float32)
        esc = jnp.zeros_like(scores)
        qej = qe[sl]
        for t in range(5):
            esc = jnp.where(et == t,
                            jnp.broadcast_to(qej[:, t:t + 1], scores.shape),
                            esc)
        scores = (scores + esc) * isq
        scores = jnp.where(adj > 0, scores, neg)
        mx = jnp.max(scores, axis=-1, keepdims=True)
        p = jnp.exp(scores - mx)
        attn = p / jnp.sum(p, axis=-1, keepdims=True)
        row_has = (jnp.sum(adj.astype(jnp.float32), axis=-1, keepdims=True)
                   > 0.0).astype(jnp.float32)
        attn = attn * row_has
        outs.append(jnp.dot(attn, v, preferred_element_type=jnp.float32))
    hidden = jnp.concatenate(outs, axis=0) + h_ref[...]
    for j in range(_GPB):
        hidden_ref[j] = hidden[j * S:(j + 1) * S]
    kl = jnp.dot(hidden, W_kl_ref[...],
                 preferred_element_type=jnp.float32) + b_kl_ref[...]
    mu = jnp.mean(kl, axis=-1, keepdims=True)
    var = jnp.mean((kl - mu) ** 2, axis=-1, keepdims=True)
    key = ((kl - mu) / jnp.sqrt(var + 1e-5)) * g_k_ref[...] + beta_k_ref[...]
    pad = (jnp.sum(mask_ref[...], axis=-1) != 0.0).astype(jnp.float32)
    for j in range(_GPB):
        sl = slice(j * S, (j + 1) * S)
        logits = jnp.sum(key[sl] * query_ref[j:j + 1, :], axis=-1)
        recall_ref[j:j + 1, :] = (jax.nn.sigmoid(logits) * pad[sl])[None, :]


def _attn(h, q, k, v, adj, et, ew, query, mask, W_kl, b_kl, g_k, beta_k):
    B = adj.shape[0]
    S = adj.shape[1]
    L = mask.shape[-1]
    R = _GPB * S
    nstep = B // _GPB
    rows = pl.BlockSpec((R, D), lambda i: (i, 0))
    bss = pl.BlockSpec((_GPB, S, S), lambda i: (i, 0, 0))
    full = lambda shape: pl.BlockSpec(shape, lambda i: tuple(0 for _ in shape))
    outs = (
        jax.ShapeDtypeStruct((B, S, D), jnp.float32),
        jax.ShapeDtypeStruct((B, S), jnp.float32),
    )
    return pl.pallas_call(
        functools.partial(_attn_body, S=S),
        grid=(nstep,),
        in_specs=[rows, rows, rows, rows, bss, bss,
                  full((5, D)),
                  pl.BlockSpec((_GPB, D), lambda i: (i, 0)),
                  pl.BlockSpec((R, L), lambda i: (i, 0)),
                  full((D, D)), full((1, D)), full((1, D)), full((1, D))],
        out_specs=[pl.BlockSpec((_GPB, S, D), lambda i: (i, 0, 0)),
                   pl.BlockSpec((_GPB, S), lambda i: (i, 0))],
        out_shape=outs,
    )(h, q, k, v, adj, et, ew, query, mask, W_kl, b_kl, g_k, beta_k)


def kernel(sentences_hidden, sentences_num, sentences_mask,
           sent_adjacent_matrix, head_type, edge_type, node_query,
           W_hp, b_hp, W_ql, b_ql, W_kl, b_kl, g_q, beta_q, g_k, beta_k,
           flag_embed, edge_embed, Wq, Wk, Wv, We):
    BS, L, DH = sentences_hidden.shape
    B = sentences_num.shape[0]
    S = BS // B

    s3 = sentences_hidden.reshape(BS, L, DH)
    ps_sc = _pool_sc(s3)
    ps_tc = _pool_tc_tail(s3)

    head_flat = head_type.reshape(BS, 1).astype(jnp.int32)
    r1 = lambda x: x.reshape(1, -1)
    h, q, k, v, ew, query = _dense(
        ps_sc, ps_tc, sentences_mask, head_flat, node_query, W_hp, r1(b_hp),
        W_ql, r1(b_ql), r1(g_q), r1(beta_q), flag_embed, edge_embed,
        Wq, Wk, Wv, We)

    adj = sent_adjacent_matrix.astype(jnp.int32)
    et = edge_type.astype(jnp.int32)
    hidden, recall = _attn(h, q, k, v, adj, et, ew, query,
                           sentences_mask,
                           W_kl, r1(b_kl), r1(g_k), r1(beta_k))
    return recall, hidden


# speedup vs baseline: 4.0193x; 1.0763x over previous
import functools
import math

import jax
import jax.numpy as jnp
from jax import lax
from jax.experimental import pallas as pl
from jax.experimental.pallas import tpu as pltpu
from jax.experimental.pallas import tpu_sc as plsc

D = 512

_NW = 32
_R_SC = 512
_RPW = _R_SC // _NW
_LTOK = 64
_DH = 768
_NCH = _DH // 16


def _pool_sc_body(s_hbm, out_hbm, buf_v, outv_v, sem):
    wid = lax.axis_index("s") * 2 + lax.axis_index("c")
    base = wid * _RPW

    pltpu.async_copy(s_hbm.at[base], buf_v.at[pl.ds(0, _LTOK)], sem)

    def sent_body(i, carry):
        p = lax.rem(i, 2) * _LTOK
        row = base + i
        pltpu.make_async_copy(
            s_hbm.at[row], buf_v.at[pl.ds(p, _LTOK)], sem).wait()

        @pl.when(i + 1 < _RPW)
        def _():
            pn = lax.rem(i + 1, 2) * _LTOK
            pltpu.async_copy(s_hbm.at[row + 1],
                             buf_v.at[pl.ds(pn, _LTOK)], sem)

        G = 12
        for g in range(_NCH // G):
            def tok_body(l2, accs, g=g):
                l = p + 2 * l2
                a = tuple(
                    accs[j] + buf_v[l, pl.ds(16 * (G * g + j), 16)]
                    for j in range(G))
                return tuple(
                    a[j] + buf_v[l + 1, pl.ds(16 * (G * g + j), 16)]
                    for j in range(G))
            z = jnp.zeros((16,), jnp.float32)
            accs = lax.fori_loop(0, _LTOK // 2, tok_body, (z,) * G)
            for j in range(G):
                outv_v[i, pl.ds(16 * (G * g + j), 16)] = accs[j]
        return carry

    lax.fori_loop(0, _RPW, sent_body, 0)
    pltpu.sync_copy(outv_v, out_hbm.at[pl.ds(base, _RPW)])


def _pool_sc(sentences_hidden3):
    mesh = plsc.VectorSubcoreMesh(core_axis_name="c", subcore_axis_name="s")
    kfn = pl.kernel(
        _pool_sc_body,
        mesh=mesh,
        out_type=jax.ShapeDtypeStruct((_R_SC, _DH), jnp.float32),
        scratch_types=[
            pltpu.VMEM((2 * _LTOK, _DH), jnp.float32),
            pltpu.VMEM((_RPW, _DH), jnp.float32),
            pltpu.SemaphoreType.DMA,
        ],
    )
    return kfn(sentences_hidden3)


def _pool_body(s_ref, out_ref):
    out_ref[...] = jnp.sum(s_ref[...], axis=1)


def _pool_tc_tail(sentences_hidden3, rows_per_block=16):
    BS, L, DH = sentences_hidden3.shape
    ntail = BS - _R_SC
    nblk = ntail // rows_per_block
    off = _R_SC // rows_per_block
    return pl.pallas_call(
        _pool_body,
        grid=(nblk,),
        in_specs=[
            pl.BlockSpec((rows_per_block, L, DH), lambda i: (i + off, 0, 0)),
        ],
        out_specs=pl.BlockSpec((rows_per_block, DH), lambda i: (i, 0)),
        out_shape=jax.ShapeDtypeStruct((ntail, DH), jnp.float32),
    )(sentences_hidden3)


def _fused_body(ps_sc_ref, ps_tc_ref, mask_ref, ht_ref, nq_ref, adj_ref,
                et_ref, W_hp_ref, b_hp_ref, W_ql_ref, b_ql_ref, W_kl_ref,
                b_kl_ref, g_q_ref, beta_q_ref, g_k_ref, beta_k_ref,
                flag_ref, edge_ref, Wq_ref, Wk_ref, Wv_ref, We_ref,
                hidden_ref, recall_ref, *, B, S):
    ps = jnp.concatenate([ps_sc_ref[...], ps_tc_ref[...]], axis=0)
    msum = jnp.sum(mask_ref[...], axis=1, keepdims=True)
    inv = 1.0 / jnp.where(msum != 0.0, msum, 1.0)
    node = jnp.dot(ps, W_hp_ref[...],
                   preferred_element_type=jnp.float32) * inv + b_hp_ref[...]
    ht = ht_ref[...].astype(jnp.float32)
    f0 = flag_ref[0:1, :]
    f1 = flag_ref[1:2, :]
    h = node + f0 + ht * (f1 - f0)
    q = jnp.dot(h, Wq_ref[...], preferred_element_type=jnp.float32)
    k = jnp.dot(h, Wk_ref[...], preferred_element_type=jnp.float32)
    v = jnp.dot(h, Wv_ref[...], preferred_element_type=jnp.float32)
    ew = jnp.dot(edge_ref[...], We_ref[...],
                 preferred_element_type=jnp.float32)
    ql = jnp.dot(nq_ref[...], W_ql_ref[...],
                 preferred_element_type=jnp.float32) + b_ql_ref[...]
    mu = jnp.mean(ql, axis=-1, keepdims=True)
    var = jnp.mean((ql - mu) ** 2, axis=-1, keepdims=True)
    query = ((ql - mu) / jnp.sqrt(var + 1e-5)) * g_q_ref[...] \
        + beta_q_ref[...]

    dn = (((1,), (1,)), ((), ()))
    qe = lax.dot_general(q, ew, dn,
                         preferred_element_type=jnp.float32)
    isq = 1.0 / math.sqrt(float(D))
    neg = jnp.float32(-1e9)
    outs = []
    for j in range(B):
        sl = slice(j * S, (j + 1) * S)
        adj = adj_ref[j]
        et = et_ref[j]
        scores = lax.dot_general(q[sl], k[sl], dn,
                                 preferred_element_type=jnp.float32)
        esc = jnp.zeros_like(scores)
        qej = qe[sl]
        for t in range(5):
            esc = jnp.where(et == t,
                            jnp.broadcast_to(qej[:, t:t + 1], scores.shape),
                            esc)
        scores = (scores + esc) * isq
        scores = jnp.where(adj > 0, scores, neg)
        mx = jnp.max(scores, axis=-1, keepdims=True)
        p = jnp.exp(scores - mx)
        attn = p / jnp.sum(p, axis=-1, keepdims=True)
        row_has = (jnp.sum(adj.astype(jnp.float32), axis=-1, keepdims=True)
                   > 0.0).astype(jnp.float32)
        attn = attn * row_has
        outs.append(jnp.dot(attn, v[sl],
                            preferred_element_type=jnp.float32))
    hidden = jnp.concatenate(outs, axis=0) + h
    for j in range(B):
        hidden_ref[j] = hidden[j * S:(j + 1) * S]
    kl = jnp.dot(hidden, W_kl_ref[...],
                 preferred_element_type=jnp.float32) + b_kl_ref[...]
    mu = jnp.mean(kl, axis=-1, keepdims=True)
    var = jnp.mean((kl - mu) ** 2, axis=-1, keepdims=True)
    key = ((kl - mu) / jnp.sqrt(var + 1e-5)) * g_k_ref[...] + beta_k_ref[...]
    pad = (jnp.sum(mask_ref[...], axis=-1) != 0.0).astype(jnp.float32)
    for j in range(B):
        sl = slice(j * S, (j + 1) * S)
        logits = jnp.sum(key[sl] * query[j:j + 1, :], axis=-1)
        recall_ref[j:j + 1, :] = (jax.nn.sigmoid(logits) * pad[sl])[None, :]


def _fused(ps_sc, ps_tc, mask, head_flat, node_query, adj, et,
           W_hp, b_hp, W_ql, b_ql, W_kl, b_kl, g_q, beta_q, g_k, beta_k,
           flag_embed, edge_embed, Wq, Wk, Wv, We):
    B = adj.shape[0]
    S = adj.shape[1]
    outs = (
        jax.ShapeDtypeStruct((B, S, D), jnp.float32),
        jax.ShapeDtypeStruct((B, S), jnp.float32),
    )
    return pl.pallas_call(
        functools.partial(_fused_body, B=B, S=S), out_shape=outs)(
        ps_sc, ps_tc, mask, head_flat, node_query, adj, et, W_hp, b_hp,
        W_ql, b_ql, W_kl, b_kl, g_q, beta_q, g_k, beta_k, flag_embed,
        edge_embed, Wq, Wk, Wv, We)


def kernel(sentences_hidden, sentences_num, sentences_mask,
           sent_adjacent_matrix, head_type, edge_type, node_query,
           W_hp, b_hp, W_ql, b_ql, W_kl, b_kl, g_q, beta_q, g_k, beta_k,
           flag_embed, edge_embed, Wq, Wk, Wv, We):
    BS, L, DH = sentences_hidden.shape
    B = sentences_num.shape[0]
    S = BS // B

    s3 = sentences_hidden.reshape(BS, L, DH)
    ps_sc = _pool_sc(s3)
    ps_tc = _pool_tc_tail(s3)

    head_flat = head_type.reshape(BS, 1).astype(jnp.int32)
    r1 = lambda x: x.reshape(1, -1)
    adj = sent_adjacent_matrix.astype(jnp.int32)
    et = edge_type.astype(jnp.int32)
    hidden, recall = _fused(
        ps_sc, ps_tc, sentences_mask, head_flat, node_query, adj, et,
        W_hp, r1(b_hp), W_ql, r1(b_ql), W_kl, r1(b_kl), r1(g_q), r1(beta_q),
        r1(g_k), r1(beta_k), flag_embed, edge_embed, Wq, Wk, Wv, We)
    return recall, hidden
